# Initial kernel scaffold; baseline (speedup 1.0000x reference)
import functools

import jax
import jax.numpy as jnp
from jax import lax
from jax.experimental import pallas as pl
from jax.experimental.pallas import tpu as pltpu
from jax.experimental.pallas import tpu_sc as plsc

_NC = 2
_NS = 16
_L = 16

_CHUNK = 128


@functools.lru_cache(maxsize=None)
def _make_sc_spmm(N, D, R, EPTP):
    assert R % _NC == 0 and N % _NS == 0 and EPTP % _CHUNK == 0
    rpc = R // _NC
    n_chunks = EPTP // _CHUNK
    rpt = N // _NS
    mesh = plsc.VectorSubcoreMesh(core_axis_name="c", subcore_axis_name="s")

    @functools.partial(
        pl.kernel,
        out_type=jax.ShapeDtypeStruct((R * N, D), jnp.float32),
        mesh=mesh,
        scratch_types=[
            pltpu.VMEM((n_chunks, _CHUNK), jnp.int32),
            pltpu.VMEM((n_chunks, _CHUNK), jnp.int32),
            pltpu.VMEM((n_chunks, _CHUNK), jnp.float32),
            pltpu.VMEM((_CHUNK, D), jnp.float32),
            pltpu.VMEM((_CHUNK, D), jnp.float32),
            pltpu.VMEM_SHARED((N, D), jnp.float32),
            pltpu.SemaphoreType.DMA,
        ],
    )
    def spmm(x_hbm, src_hbm, dst_hbm, w_hbm, out_hbm,
             src_v, dst_v, w_v, rows_v, zero_v, acc, sem):
        cid = lax.axis_index("c")
        sid = lax.axis_index("s")

        def _zb(i, carry):
            z = jnp.zeros((_L,), jnp.float32)
            for j in range(D // _L):
                zero_v[i, pl.ds(j * _L, _L)] = z
            return carry
        lax.fori_loop(0, _CHUNK, _zb, 0)

        for rr in range(rpc):
            r = cid * rpc + rr

            row0 = sid * rpt
            done = 0
            while done < rpt:
                nrows = min(_CHUNK, rpt - done)
                pltpu.sync_copy(zero_v.at[pl.ds(0, nrows)],
                                acc.at[pl.ds(row0 + done, nrows)])
                done += nrows
            plsc.subcore_barrier()

            seg = r * _NS + sid
            pltpu.sync_copy(src_hbm.at[seg], src_v)
            pltpu.sync_copy(dst_hbm.at[seg], dst_v)
            pltpu.sync_copy(w_hbm.at[seg], w_v)

            def _chunk(k, carry):
                pltpu.async_copy(x_hbm.at[src_v.at[k]], rows_v, sem).wait()

                def _scale(e, c2):
                    w = w_v[k, e]
                    for j in range(D // _L):
                        sl = pl.ds(j * _L, _L)
                        rows_v[e, sl] = rows_v[e, sl] * w
                    return c2
                lax.fori_loop(0, _CHUNK, _scale, 0)

                pltpu.sync_copy(rows_v, acc.at[dst_v.at[k]], add=True)
                return carry
            lax.fori_loop(0, n_chunks, _chunk, 0)
            plsc.subcore_barrier()

            done = 0
            while done < rpt:
                nrows = min(_CHUNK, rpt - done)
                pltpu.sync_copy(acc.at[pl.ds(row0 + done, nrows)],
                                out_hbm.at[pl.ds(r * N + row0 + done, nrows)])
                done += nrows

    return spmm


@functools.lru_cache(maxsize=None)
def _make_dense(N, D, R, DOUT, B):
    assert N % B == 0
    grid = (N // B,)

    def body(s_ref, w_ref, sw_ref, b_ref, g_ref, be_ref, o_ref):
        ssum = s_ref[0]
        for r in range(1, R):
            ssum = ssum + s_ref[r]
        mu = jnp.mean(ssum, axis=-1, keepdims=True)
        d = ssum - mu
        var = jnp.mean(d * d, axis=-1, keepdims=True)
        sn = d * lax.rsqrt(var + 1e-6) * g_ref[...] + be_ref[...]
        acc = jnp.dot(sn, sw_ref[...], preferred_element_type=jnp.float32)
        for r in range(R):
            acc = acc + jnp.dot(s_ref[r], w_ref[r],
                                preferred_element_type=jnp.float32)
        o_ref[...] = acc * 0.5 + b_ref[...]

    return pl.pallas_call(
        body,
        grid=grid,
        in_specs=[
            pl.BlockSpec((R, B, D), lambda i: (0, i, 0)),
            pl.BlockSpec((R, D, DOUT), lambda i: (0, 0, 0)),
            pl.BlockSpec((D, DOUT), lambda i: (0, 0)),
            pl.BlockSpec((1, DOUT), lambda i: (0, 0)),
            pl.BlockSpec((1, D), lambda i: (0, 0)),
            pl.BlockSpec((1, D), lambda i: (0, 0)),
        ],
        out_specs=pl.BlockSpec((B, DOUT), lambda i: (i, 0)),
        out_shape=jax.ShapeDtypeStruct((N, DOUT), jnp.float32),
    )


def kernel(x, edge_index, edge_weight, weight, share_weight, bias,
           ln_gamma, ln_beta):
    N, D = x.shape
    R, _, E = edge_index.shape
    DOUT = weight.shape[1]

    ept = E // _NS
    n_chunks = -(-ept // _CHUNK)
    eptp = n_chunks * _CHUNK
    pad = eptp - ept

    src = edge_index[:, 0, :].reshape(R, _NS, ept)
    dst = edge_index[:, 1, :].reshape(R, _NS, ept)
    ew = edge_weight.reshape(R, _NS, ept)
    if pad:
        src = jnp.pad(src, ((0, 0), (0, 0), (0, pad)))
        dst = jnp.pad(dst, ((0, 0), (0, 0), (0, pad)))
        ew = jnp.pad(ew, ((0, 0), (0, 0), (0, pad)))
    src3 = src.reshape(R * _NS, n_chunks, _CHUNK)
    dst3 = dst.reshape(R * _NS, n_chunks, _CHUNK)
    w3 = ew.reshape(R * _NS, n_chunks, _CHUNK)

    supports = _make_sc_spmm(N, D, R, eptp)(x, src3, dst3, w3)
    s = supports.reshape(R, N, D)

    dense = _make_dense(N, D, R, DOUT, B=1000)
    return dense(
        s,
        weight.reshape(R, D, DOUT),
        share_weight,
        bias.reshape(1, DOUT),
        ln_gamma.reshape(1, D),
        ln_beta.reshape(1, D),
    )


# speedup vs baseline: 3.1386x; 3.1386x over previous
import functools

import jax
import jax.numpy as jnp
from jax import lax
from jax.experimental import pallas as pl
from jax.experimental.pallas import tpu as pltpu
from jax.experimental.pallas import tpu_sc as plsc

_NC = 2
_NS = 16
_L = 16

_CHUNK = 128


@functools.lru_cache(maxsize=None)
def _make_sc_spmm(N, D, R, EPTP, NPAD):
    assert R % _NC == 0 and NPAD % (8 * _NS) == 0 and EPTP % _CHUNK == 0
    rpc = R // _NC
    n_chunks = EPTP // _CHUNK
    rpt = NPAD // _NS
    mesh = plsc.VectorSubcoreMesh(core_axis_name="c", subcore_axis_name="s")

    @functools.partial(
        pl.kernel,
        out_type=jax.ShapeDtypeStruct((R * NPAD, D), jnp.float32),
        mesh=mesh,
        scratch_types=[
            pltpu.VMEM((2, _CHUNK), jnp.int32),
            pltpu.VMEM((_CHUNK,), jnp.float32),
            pltpu.VMEM((_CHUNK, D), jnp.float32),
            pltpu.VMEM_SHARED((NPAD, D), jnp.float32),
            pltpu.SemaphoreType.DMA,
        ],
    )
    def spmm(x_hbm, packed_hbm, w_hbm, out_hbm, packed_v, w_v, rows_v, acc,
             sem):
        cid = lax.axis_index("c")
        sid = lax.axis_index("s")
        row0 = sid * rpt

        for rr in range(rpc):
            r = cid * rpc + rr

            def _zb(i, carry):
                z = jnp.zeros((_L,), jnp.float32)
                for j in range(D // _L):
                    rows_v[i, pl.ds(j * _L, _L)] = z
                return carry
            lax.fori_loop(0, _CHUNK, _zb, 0)

            done = 0
            while done < rpt:
                nrows = min(_CHUNK, rpt - done)
                pltpu.sync_copy(rows_v.at[pl.ds(0, nrows)],
                                acc.at[pl.ds(row0 + done, nrows)])
                done += nrows
            plsc.subcore_barrier()

            seg = (r * _NS + sid) * n_chunks

            def _chunk(k, carry):
                pltpu.sync_copy(packed_hbm.at[seg + k], packed_v)
                pltpu.sync_copy(w_hbm.at[seg + k], w_v)
                pltpu.async_copy(x_hbm.at[packed_v.at[0]], rows_v, sem).wait()

                def _scale(c16, c2):
                    wvec = w_v[pl.ds(c16 * _L, _L)]
                    for e16 in range(_L):
                        w = wvec[e16]
                        e = c16 * _L + e16
                        for j in range(D // _L):
                            sl = pl.ds(j * _L, _L)
                            rows_v[e, sl] = rows_v[e, sl] * w
                    return c2
                lax.fori_loop(0, _CHUNK // _L, _scale, 0)

                pltpu.sync_copy(rows_v, acc.at[packed_v.at[1]], add=True)
                return carry
            lax.fori_loop(0, n_chunks, _chunk, 0)
            plsc.subcore_barrier()

            done = 0
            while done < rpt:
                nrows = min(_CHUNK, rpt - done)
                pltpu.sync_copy(acc.at[pl.ds(row0 + done, nrows)],
                                out_hbm.at[pl.ds(r * NPAD + row0 + done,
                                                 nrows)])
                done += nrows

    return spmm


@functools.lru_cache(maxsize=None)
def _make_dense(N, D, R, DOUT, B):
    assert N % B == 0
    grid = (N // B,)

    def body(s_ref, w_ref, sw_ref, b_ref, g_ref, be_ref, o_ref):
        ssum = s_ref[0]
        for r in range(1, R):
            ssum = ssum + s_ref[r]
        mu = jnp.mean(ssum, axis=-1, keepdims=True)
        d = ssum - mu
        var = jnp.mean(d * d, axis=-1, keepdims=True)
        sn = d * lax.rsqrt(var + 1e-6) * g_ref[...] + be_ref[...]
        acc = jnp.dot(sn, sw_ref[...], preferred_element_type=jnp.float32)
        for r in range(R):
            acc = acc + jnp.dot(s_ref[r], w_ref[r],
                                preferred_element_type=jnp.float32)
        o_ref[...] = acc * 0.5 + b_ref[...]

    return pl.pallas_call(
        body,
        grid=grid,
        in_specs=[
            pl.BlockSpec((R, B, D), lambda i: (0, i, 0)),
            pl.BlockSpec((R, D, DOUT), lambda i: (0, 0, 0)),
            pl.BlockSpec((D, DOUT), lambda i: (0, 0)),
            pl.BlockSpec((1, DOUT), lambda i: (0, 0)),
            pl.BlockSpec((1, D), lambda i: (0, 0)),
            pl.BlockSpec((1, D), lambda i: (0, 0)),
        ],
        out_specs=pl.BlockSpec((B, DOUT), lambda i: (i, 0)),
        out_shape=jax.ShapeDtypeStruct((N, DOUT), jnp.float32),
    )


def kernel(x, edge_index, edge_weight, weight, share_weight, bias,
           ln_gamma, ln_beta):
    N, D = x.shape
    R, _, E = edge_index.shape
    DOUT = weight.shape[1]

    ept = E // _NS
    n_chunks = -(-ept // _CHUNK)
    eptp = n_chunks * _CHUNK
    pad = eptp - ept

    src = edge_index[:, 0, :].reshape(R, _NS, ept)
    dst = edge_index[:, 1, :].reshape(R, _NS, ept)
    ew = edge_weight.reshape(R, _NS, ept)
    if pad:
        src = jnp.pad(src, ((0, 0), (0, 0), (0, pad)))
        dst = jnp.pad(dst, ((0, 0), (0, 0), (0, pad)))
        ew = jnp.pad(ew, ((0, 0), (0, 0), (0, pad)))
    packed = jnp.stack(
        [src.reshape(R * _NS, n_chunks, _CHUNK),
         dst.reshape(R * _NS, n_chunks, _CHUNK)], axis=2)
    packed = packed.reshape(R * _NS * n_chunks, 2, _CHUNK)
    wchunks = ew.reshape(R * _NS * n_chunks, _CHUNK)

    npad = -(-N // (8 * _NS)) * (8 * _NS)
    supports = _make_sc_spmm(N, D, R, eptp, npad)(x, packed, wchunks)
    s = supports.reshape(R, npad, D)

    dense = _make_dense(N, D, R, DOUT, B=1000)
    return dense(
        s,
        weight.reshape(R, D, DOUT),
        share_weight,
        bias.reshape(1, DOUT),
        ln_gamma.reshape(1, D),
        ln_beta.reshape(1, D),
    )
